# Initial kernel scaffold; baseline (speedup 1.0000x reference)
import functools

import jax
import jax.numpy as jnp
from jax import lax
from jax.experimental import pallas as pl
from jax.experimental.pallas import tpu as pltpu
from jax.experimental.pallas import tpu_sc as plsc

N = 10000
E = 320000
D = 128
DE = 16
NEG = 0.2

NC, NS, L = 2, 16, 16
NW = NC * NS
EPW = E // NW
B = 125
NBLK = EPW // B
RPT = N // NS

_PREC = lax.Precision.HIGHEST


def _leaky(z):
    return jnp.maximum(z, 0.0) + NEG * jnp.minimum(z, 0.0)


def _dot(a, b):
    return jnp.dot(a, b, precision=_PREC, preferred_element_type=jnp.float32)


def _prep_body(x_ref, w_ref, b_ref, o_ref):
    o_ref[...] = _leaky(_dot(x_ref[...], w_ref[...]) + b_ref[...])


def _prep(x, Wp, bp):
    return pl.pallas_call(
        _prep_body,
        grid=(10,),
        in_specs=[
            pl.BlockSpec((N // 10, D), lambda i: (i, 0)),
            pl.BlockSpec((D, D), lambda i: (0, 0)),
            pl.BlockSpec((1, D), lambda i: (0, 0)),
        ],
        out_specs=pl.BlockSpec((N // 10, D), lambda i: (i, 0)),
        out_shape=jax.ShapeDtypeStruct((N, D), jnp.float32),
    )(x, Wp, bp.reshape(1, D))


def _edge_body(ew_ref, w_ref, b_ref, o_ref):
    o_ref[...] = _dot(ew_ref[...], w_ref[...]) + b_ref[...]


def _edge_lin(ew, We, be):
    EB = 8000
    return pl.pallas_call(
        _edge_body,
        grid=(E // EB,),
        in_specs=[
            pl.BlockSpec((EB, DE), lambda i: (i, 0)),
            pl.BlockSpec((DE, D), lambda i: (0, 0)),
            pl.BlockSpec((1, D), lambda i: (0, 0)),
        ],
        out_specs=pl.BlockSpec((EB, D), lambda i: (i, 0)),
        out_shape=jax.ShapeDtypeStruct((E, D), jnp.float32),
    )(ew, We, be.reshape(1, D))


def _mlp_body(h_ref, p0_ref, p1_ref, w1_ref, b1_ref, w2_ref, b2_ref, o_ref):
    z = p0_ref[0] + p1_ref[0] - h_ref[...]
    a = _leaky(_dot(z, w1_ref[...]) + b1_ref[...])
    o_ref[...] = jnp.tanh(_dot(a, w2_ref[...]) + b2_ref[...])


def _mlp(h, P, W1, b1, W2, b2):
    NB = N // 10
    return pl.pallas_call(
        _mlp_body,
        grid=(10,),
        in_specs=[
            pl.BlockSpec((NB, D), lambda i: (i, 0)),
            pl.BlockSpec((1, NB, D), lambda i: (0, i, 0)),
            pl.BlockSpec((1, NB, D), lambda i: (1, i, 0)),
            pl.BlockSpec((D, D), lambda i: (0, 0)),
            pl.BlockSpec((1, D), lambda i: (0, 0)),
            pl.BlockSpec((D, D), lambda i: (0, 0)),
            pl.BlockSpec((1, D), lambda i: (0, 0)),
        ],
        out_specs=pl.BlockSpec((NB, D), lambda i: (i, 0)),
        out_shape=jax.ShapeDtypeStruct((N, D), jnp.float32),
    )(h, P, P, W1, b1.reshape(1, D), W2, b2.reshape(1, D))


def _mlp_post_body(h_ref, p0_ref, p1_ref, w1_ref, b1_ref, w2_ref, b2_ref,
                   wp_ref, bp_ref, o_ref):
    z = p0_ref[0] + p1_ref[0] - h_ref[...]
    a = _leaky(_dot(z, w1_ref[...]) + b1_ref[...])
    t = jnp.tanh(_dot(a, w2_ref[...]) + b2_ref[...])
    o_ref[...] = jnp.tanh(_dot(t, wp_ref[...]) + bp_ref[...])


def _mlp_post(h, P, W1, b1, W2, b2, Wpost, bpost):
    NB = N // 10
    return pl.pallas_call(
        _mlp_post_body,
        grid=(10,),
        in_specs=[
            pl.BlockSpec((NB, D), lambda i: (i, 0)),
            pl.BlockSpec((1, NB, D), lambda i: (0, i, 0)),
            pl.BlockSpec((1, NB, D), lambda i: (1, i, 0)),
            pl.BlockSpec((D, D), lambda i: (0, 0)),
            pl.BlockSpec((1, D), lambda i: (0, 0)),
            pl.BlockSpec((D, D), lambda i: (0, 0)),
            pl.BlockSpec((1, D), lambda i: (0, 0)),
            pl.BlockSpec((D, D), lambda i: (0, 0)),
            pl.BlockSpec((1, D), lambda i: (0, 0)),
        ],
        out_specs=pl.BlockSpec((NB, D), lambda i: (i, 0)),
        out_shape=jax.ShapeDtypeStruct((N, D), jnp.float32),
    )(h, P, P, W1, b1.reshape(1, D), W2, b2.reshape(1, D), Wpost,
      bpost.reshape(1, D))


def _sc_aggr(h, e, src2, dst2):
    mesh = plsc.VectorSubcoreMesh(core_axis_name="c", subcore_axis_name="s")

    @functools.partial(
        pl.kernel,
        out_type=jax.ShapeDtypeStruct((NC, N, D), jnp.float32),
        mesh=mesh,
        scratch_types=[
            pltpu.VMEM((NBLK, B), jnp.int32),
            pltpu.VMEM((NBLK, B), jnp.int32),
            pltpu.VMEM((B, D), jnp.float32),
            pltpu.VMEM((B, D), jnp.float32),
            pltpu.VMEM((B, D), jnp.float32),
            pltpu.VMEM((B, D), jnp.float32),
            pltpu.VMEM_SHARED((N, D), jnp.float32),
            pltpu.SemaphoreType.DMA,
            pltpu.SemaphoreType.DMA,
            pltpu.SemaphoreType.DMA,
            pltpu.SemaphoreType.DMA,
        ],
    )
    def k(h_hbm, e_hbm, src_hbm, dst_hbm, out_hbm,
          sidx, didx, eb0, eb1, gb0, gb1, acc, se0, se1, sg0, sg1):
        cid = lax.axis_index("c")
        sid = lax.axis_index("s")
        wid = sid * NC + cid
        ebase = wid * EPW
        ibase = wid * NBLK
        r0 = sid * RPT

        ebufs = (eb0, eb1)
        gbufs = (gb0, gb1)
        esems = (se0, se1)
        gsems = (sg0, sg1)

        pltpu.sync_copy(h_hbm.at[pl.ds(r0, RPT)], acc.at[pl.ds(r0, RPT)])
        pltpu.sync_copy(src_hbm.at[pl.ds(ibase, NBLK)], sidx)
        pltpu.sync_copy(dst_hbm.at[pl.ds(ibase, NBLK)], didx)
        plsc.subcore_barrier()

        def issue(blk, ph):
            pltpu.async_copy(
                e_hbm.at[pl.ds(ebase + blk * B, B)], ebufs[ph], esems[ph])
            pltpu.async_copy(h_hbm.at[sidx.at[blk]], gbufs[ph], gsems[ph])

        issue(0, 0)
        issue(1, 1)

        @pl.loop(0, NBLK, step=2)
        def _(blk0):
            for ph in range(2):
                blk = blk0 + ph
                eb, gb = ebufs[ph], gbufs[ph]
                pltpu.make_async_copy(
                    e_hbm.at[pl.ds(ebase + blk * B, B)], eb, esems[ph]).wait()
                pltpu.make_async_copy(
                    h_hbm.at[sidx.at[blk]], gb, gsems[ph]).wait()

                @pl.loop(0, B)
                def _(r):
                    for k8 in range(D // L):
                        sl = pl.ds(k8 * L, L)
                        eb[r, sl] = jnp.maximum(eb[r, sl] + gb[r, sl], 0.0)

                pltpu.sync_copy(eb, acc.at[didx.at[blk]], add=True)

                @pl.when(blk + 2 < NBLK)
                def _():
                    issue(blk + 2, ph)

        plsc.subcore_barrier()
        pltpu.sync_copy(acc.at[pl.ds(r0, RPT)],
                        out_hbm.at[cid, pl.ds(r0, RPT)])

    return k(h, e, src2, dst2)


def kernel(x, edge_index, edge_weight, Wp, bp, We0, be0, W01, b01, W02, b02,
           We1, be1, W11, b11, W12, b12, Wpost, bpost):
    src2 = edge_index[0].astype(jnp.int32).reshape(NW * NBLK, B)
    dst2 = edge_index[1].astype(jnp.int32).reshape(NW * NBLK, B)

    h = _prep(x, Wp, bp)
    E0 = _edge_lin(edge_weight, We0, be0)
    E1 = _edge_lin(edge_weight, We1, be1)

    P0 = _sc_aggr(h, E0, src2, dst2)
    h1 = _mlp(h, P0, W01, b01, W02, b02)

    P1 = _sc_aggr(h1, E1, src2, dst2)
    out = _mlp_post(h1, P1, W11, b11, W12, b12, Wpost, bpost)
    return out


# speedup vs baseline: 4.3892x; 4.3892x over previous
import functools

import jax
import jax.numpy as jnp
from jax import lax
from jax.experimental import pallas as pl
from jax.experimental.pallas import tpu as pltpu
from jax.experimental.pallas import tpu_sc as plsc

N = 10000
E = 320000
D = 128
DE = 16
NEG = 0.2

NC, NS, L = 2, 16, 16
NW = NC * NS
EPW = E // NW
B = 80
NBLK = EPW // B
RPT = 624

_PREC = None


def _leaky(z):
    return jnp.maximum(z, 0.0) + NEG * jnp.minimum(z, 0.0)


def _dot(a, b):
    return jnp.dot(a, b, precision=_PREC, preferred_element_type=jnp.float32)


def _prep_body(x_ref, w_ref, b_ref, o_ref):
    o_ref[...] = _leaky(_dot(x_ref[...], w_ref[...]) + b_ref[...])


def _prep(x, Wp, bp):
    return pl.pallas_call(
        _prep_body,
        grid=(10,),
        in_specs=[
            pl.BlockSpec((N // 10, D), lambda i: (i, 0)),
            pl.BlockSpec((D, D), lambda i: (0, 0)),
            pl.BlockSpec((1, D), lambda i: (0, 0)),
        ],
        out_specs=pl.BlockSpec((N // 10, D), lambda i: (i, 0)),
        out_shape=jax.ShapeDtypeStruct((N, D), jnp.float32),
    )(x, Wp, bp.reshape(1, D))


def _edge_body(ew_ref, w_ref, b_ref, o_ref):
    o_ref[...] = _dot(ew_ref[...], w_ref[...]) + b_ref[...]


def _edge_lin(ew, We, be):
    EB = 8000
    return pl.pallas_call(
        _edge_body,
        grid=(E // EB,),
        in_specs=[
            pl.BlockSpec((EB, DE), lambda i: (i, 0)),
            pl.BlockSpec((DE, D), lambda i: (0, 0)),
            pl.BlockSpec((1, D), lambda i: (0, 0)),
        ],
        out_specs=pl.BlockSpec((EB, D), lambda i: (i, 0)),
        out_shape=jax.ShapeDtypeStruct((E, D), jnp.float32),
    )(ew, We, be.reshape(1, D))


def _mlp_body(h_ref, p0_ref, p1_ref, w1_ref, b1_ref, w2_ref, b2_ref, o_ref):
    z = p0_ref[0] + p1_ref[0] - h_ref[...]
    a = _leaky(_dot(z, w1_ref[...]) + b1_ref[...])
    o_ref[...] = jnp.tanh(_dot(a, w2_ref[...]) + b2_ref[...])


def _mlp(h, P, W1, b1, W2, b2):
    NB = N // 10
    return pl.pallas_call(
        _mlp_body,
        grid=(10,),
        in_specs=[
            pl.BlockSpec((NB, D), lambda i: (i, 0)),
            pl.BlockSpec((1, NB, D), lambda i: (0, i, 0)),
            pl.BlockSpec((1, NB, D), lambda i: (1, i, 0)),
            pl.BlockSpec((D, D), lambda i: (0, 0)),
            pl.BlockSpec((1, D), lambda i: (0, 0)),
            pl.BlockSpec((D, D), lambda i: (0, 0)),
            pl.BlockSpec((1, D), lambda i: (0, 0)),
        ],
        out_specs=pl.BlockSpec((NB, D), lambda i: (i, 0)),
        out_shape=jax.ShapeDtypeStruct((N, D), jnp.float32),
    )(h, P, P, W1, b1.reshape(1, D), W2, b2.reshape(1, D))


def _mlp_post_body(h_ref, p0_ref, p1_ref, w1_ref, b1_ref, w2_ref, b2_ref,
                   wp_ref, bp_ref, o_ref):
    z = p0_ref[0] + p1_ref[0] - h_ref[...]
    a = _leaky(_dot(z, w1_ref[...]) + b1_ref[...])
    t = jnp.tanh(_dot(a, w2_ref[...]) + b2_ref[...])
    o_ref[...] = jnp.tanh(_dot(t, wp_ref[...]) + bp_ref[...])


def _mlp_post(h, P, W1, b1, W2, b2, Wpost, bpost):
    NB = N // 10
    return pl.pallas_call(
        _mlp_post_body,
        grid=(10,),
        in_specs=[
            pl.BlockSpec((NB, D), lambda i: (i, 0)),
            pl.BlockSpec((1, NB, D), lambda i: (0, i, 0)),
            pl.BlockSpec((1, NB, D), lambda i: (1, i, 0)),
            pl.BlockSpec((D, D), lambda i: (0, 0)),
            pl.BlockSpec((1, D), lambda i: (0, 0)),
            pl.BlockSpec((D, D), lambda i: (0, 0)),
            pl.BlockSpec((1, D), lambda i: (0, 0)),
            pl.BlockSpec((D, D), lambda i: (0, 0)),
            pl.BlockSpec((1, D), lambda i: (0, 0)),
        ],
        out_specs=pl.BlockSpec((NB, D), lambda i: (i, 0)),
        out_shape=jax.ShapeDtypeStruct((N, D), jnp.float32),
    )(h, P, P, W1, b1.reshape(1, D), W2, b2.reshape(1, D), Wpost,
      bpost.reshape(1, D))


def _sc_aggr(h, e, src2, dst2):
    mesh = plsc.VectorSubcoreMesh(core_axis_name="c", subcore_axis_name="s")

    @functools.partial(
        pl.kernel,
        out_type=jax.ShapeDtypeStruct((NC, N, D), jnp.float32),
        mesh=mesh,
        scratch_types=[
            pltpu.VMEM((1, B), jnp.int32),
            pltpu.VMEM((1, B), jnp.int32),
            pltpu.VMEM((1, B), jnp.int32),
            pltpu.VMEM((1, B), jnp.int32),
            pltpu.VMEM((B, D), jnp.float32),
            pltpu.VMEM((B, D), jnp.float32),
            pltpu.VMEM((B, D), jnp.float32),
            pltpu.VMEM((B, D), jnp.float32),
            pltpu.VMEM_SHARED((N, D), jnp.float32),
            pltpu.SemaphoreType.DMA,
            pltpu.SemaphoreType.DMA,
            pltpu.SemaphoreType.DMA,
            pltpu.SemaphoreType.DMA,
            pltpu.SemaphoreType.DMA,
            pltpu.SemaphoreType.DMA,
        ],
    )
    def k(h_hbm, e_hbm, src_hbm, dst_hbm, out_hbm,
          si0, si1, di0, di1, eb0, eb1, gb0, gb1, acc,
          ssem0, ssem1, dsem0, dsem1, esem0, esem1):
        cid = lax.axis_index("c")
        sid = lax.axis_index("s")
        wid = sid * NC + cid
        ebase = wid * EPW
        r0 = sid * RPT

        sibufs = (si0, si1)
        dibufs = (di0, di1)
        ebufs = (eb0, eb1)
        gbufs = (gb0, gb1)
        sisems = (ssem0, ssem1)
        disems = (dsem0, dsem1)
        esems = (esem0, esem1)

        @pl.when(sid < NS - 1)
        def _():
            pltpu.sync_copy(h_hbm.at[pl.ds(r0, RPT)], acc.at[pl.ds(r0, RPT)])

        @pl.when(sid == NS - 1)
        def _():
            pltpu.sync_copy(h_hbm.at[pl.ds((NS - 1) * RPT, N - (NS - 1) * RPT)],
                            acc.at[pl.ds((NS - 1) * RPT, N - (NS - 1) * RPT)])

        plsc.subcore_barrier()

        def issue_idx(blk, ph):
            pltpu.async_copy(
                src_hbm.at[wid, pl.ds(blk, 1)], sibufs[ph], sisems[ph])
            pltpu.async_copy(
                dst_hbm.at[wid, pl.ds(blk, 1)], dibufs[ph], disems[ph])
            pltpu.async_copy(
                e_hbm.at[pl.ds(ebase + blk * B, B)], ebufs[ph], esems[ph])

        def wait_idx(blk, ph):
            pltpu.make_async_copy(
                src_hbm.at[wid, pl.ds(blk, 1)], sibufs[ph], sisems[ph]).wait()

        def issue_gather(blk, ph):
            pltpu.async_copy(h_hbm.at[sibufs[ph].at[0]], gbufs[ph], sisems[ph])

        def wait_data(blk, ph):
            pltpu.make_async_copy(
                e_hbm.at[pl.ds(ebase + blk * B, B)], ebufs[ph], esems[ph]).wait()
            pltpu.make_async_copy(
                h_hbm.at[sibufs[ph].at[0]], gbufs[ph], sisems[ph]).wait()

        def compute_scatter(blk, ph):
            eb, gb = ebufs[ph], gbufs[ph]

            @pl.loop(0, B)
            def _(r):
                for k8 in range(D // L):
                    sl = pl.ds(k8 * L, L)
                    eb[r, sl] = jnp.maximum(eb[r, sl] + gb[r, sl], 0.0)

            pltpu.make_async_copy(
                dst_hbm.at[wid, pl.ds(blk, 1)], dibufs[ph], disems[ph]).wait()
            pltpu.sync_copy(eb, acc.at[dibufs[ph].at[0]], add=True)

        issue_idx(0, 0)
        wait_idx(0, 0)
        issue_gather(0, 0)
        issue_idx(1, 1)
        wait_idx(1, 1)
        issue_gather(1, 1)

        @pl.loop(0, NBLK - 1, step=2)
        def _(blk0):
            for ph in range(2):
                blk = blk0 + ph
                wait_data(blk, ph)
                compute_scatter(blk, ph)

                @pl.when(blk + 2 < NBLK)
                def _():
                    issue_idx(blk + 2, ph)
                    wait_idx(blk + 2, ph)
                    issue_gather(blk + 2, ph)

        wait_data(NBLK - 1, (NBLK - 1) % 2)
        compute_scatter(NBLK - 1, (NBLK - 1) % 2)

        plsc.subcore_barrier()

        @pl.when(sid < NS - 1)
        def _():
            pltpu.sync_copy(acc.at[pl.ds(r0, RPT)],
                            out_hbm.at[cid, pl.ds(r0, RPT)])

        @pl.when(sid == NS - 1)
        def _():
            pltpu.sync_copy(acc.at[pl.ds((NS - 1) * RPT, N - (NS - 1) * RPT)],
                            out_hbm.at[cid, pl.ds((NS - 1) * RPT,
                                                  N - (NS - 1) * RPT)])

    return k(h, e, src2, dst2)


def kernel(x, edge_index, edge_weight, Wp, bp, We0, be0, W01, b01, W02, b02,
           We1, be1, W11, b11, W12, b12, Wpost, bpost):
    src2 = edge_index[0].astype(jnp.int32).reshape(NW, NBLK, B)
    dst2 = edge_index[1].astype(jnp.int32).reshape(NW, NBLK, B)

    h = _prep(x, Wp, bp)
    E0 = _edge_lin(edge_weight, We0, be0)
    E1 = _edge_lin(edge_weight, We1, be1)

    P0 = _sc_aggr(h, E0, src2, dst2)
    h1 = _mlp(h, P0, W01, b01, W02, b02)

    P1 = _sc_aggr(h1, E1, src2, dst2)
    out = _mlp_post(h1, P1, W11, b11, W12, b12, Wpost, bpost)
    return out


# speedup vs baseline: 4.7708x; 1.0869x over previous
import functools

import jax
import jax.numpy as jnp
from jax import lax
from jax.experimental import pallas as pl
from jax.experimental.pallas import tpu as pltpu
from jax.experimental.pallas import tpu_sc as plsc

N = 10000
E = 320000
D = 128
DE = 16
NEG = 0.2

NC, NS, L = 2, 16, 16
NW = NC * NS
EPW = E // NW
B = 80
NBLK = EPW // B
RPT = 624

_PREC = None


def _leaky(z):
    return jnp.maximum(z, 0.0) + NEG * jnp.minimum(z, 0.0)


def _dot(a, b):
    return jnp.dot(a, b, precision=_PREC, preferred_element_type=jnp.float32)


def _prep_body(x_ref, w_ref, b_ref, o_ref):
    o_ref[...] = _leaky(_dot(x_ref[...], w_ref[...]) + b_ref[...])


def _prep(x, Wp, bp):
    return pl.pallas_call(
        _prep_body,
        grid=(10,),
        in_specs=[
            pl.BlockSpec((N // 10, D), lambda i: (i, 0)),
            pl.BlockSpec((D, D), lambda i: (0, 0)),
            pl.BlockSpec((1, D), lambda i: (0, 0)),
        ],
        out_specs=pl.BlockSpec((N // 10, D), lambda i: (i, 0)),
        out_shape=jax.ShapeDtypeStruct((N, D), jnp.float32),
    )(x, Wp, bp.reshape(1, D))


def _edge_body(ew_ref, w_ref, b_ref, o_ref):
    o_ref[...] = _dot(ew_ref[...], w_ref[...]) + b_ref[...]


def _edge_lin(ew, We, be):
    EB = 8000
    return pl.pallas_call(
        _edge_body,
        grid=(E // EB,),
        in_specs=[
            pl.BlockSpec((EB, DE), lambda i: (i, 0)),
            pl.BlockSpec((DE, D), lambda i: (0, 0)),
            pl.BlockSpec((1, D), lambda i: (0, 0)),
        ],
        out_specs=pl.BlockSpec((EB, D), lambda i: (i, 0)),
        out_shape=jax.ShapeDtypeStruct((E, D), jnp.float32),
    )(ew, We, be.reshape(1, D))


def _mlp_body(h_ref, p0_ref, p1_ref, w1_ref, b1_ref, w2_ref, b2_ref, o_ref):
    z = p0_ref[0] + p1_ref[0] - h_ref[...]
    a = _leaky(_dot(z, w1_ref[...]) + b1_ref[...])
    o_ref[...] = jnp.tanh(_dot(a, w2_ref[...]) + b2_ref[...])


def _mlp(h, P, W1, b1, W2, b2):
    NB = N // 10
    return pl.pallas_call(
        _mlp_body,
        grid=(10,),
        in_specs=[
            pl.BlockSpec((NB, D), lambda i: (i, 0)),
            pl.BlockSpec((1, NB, D), lambda i: (0, i, 0)),
            pl.BlockSpec((1, NB, D), lambda i: (1, i, 0)),
            pl.BlockSpec((D, D), lambda i: (0, 0)),
            pl.BlockSpec((1, D), lambda i: (0, 0)),
            pl.BlockSpec((D, D), lambda i: (0, 0)),
            pl.BlockSpec((1, D), lambda i: (0, 0)),
        ],
        out_specs=pl.BlockSpec((NB, D), lambda i: (i, 0)),
        out_shape=jax.ShapeDtypeStruct((N, D), jnp.float32),
    )(h, P, P, W1, b1.reshape(1, D), W2, b2.reshape(1, D))


def _mlp_post_body(h_ref, p0_ref, p1_ref, w1_ref, b1_ref, w2_ref, b2_ref,
                   wp_ref, bp_ref, o_ref):
    z = p0_ref[0] + p1_ref[0] - h_ref[...]
    a = _leaky(_dot(z, w1_ref[...]) + b1_ref[...])
    t = jnp.tanh(_dot(a, w2_ref[...]) + b2_ref[...])
    o_ref[...] = jnp.tanh(_dot(t, wp_ref[...]) + bp_ref[...])


def _mlp_post(h, P, W1, b1, W2, b2, Wpost, bpost):
    NB = N // 10
    return pl.pallas_call(
        _mlp_post_body,
        grid=(10,),
        in_specs=[
            pl.BlockSpec((NB, D), lambda i: (i, 0)),
            pl.BlockSpec((1, NB, D), lambda i: (0, i, 0)),
            pl.BlockSpec((1, NB, D), lambda i: (1, i, 0)),
            pl.BlockSpec((D, D), lambda i: (0, 0)),
            pl.BlockSpec((1, D), lambda i: (0, 0)),
            pl.BlockSpec((D, D), lambda i: (0, 0)),
            pl.BlockSpec((1, D), lambda i: (0, 0)),
            pl.BlockSpec((D, D), lambda i: (0, 0)),
            pl.BlockSpec((1, D), lambda i: (0, 0)),
        ],
        out_specs=pl.BlockSpec((NB, D), lambda i: (i, 0)),
        out_shape=jax.ShapeDtypeStruct((N, D), jnp.float32),
    )(h, P, P, W1, b1.reshape(1, D), W2, b2.reshape(1, D), Wpost,
      bpost.reshape(1, D))


def _sc_aggr(h, e, src2, dst2):
    mesh = plsc.VectorSubcoreMesh(core_axis_name="c", subcore_axis_name="s")

    @functools.partial(
        pl.kernel,
        out_type=jax.ShapeDtypeStruct((NC, N, D), jnp.float32),
        mesh=mesh,
        scratch_types=[
            pltpu.VMEM((1, B), jnp.int32),
            pltpu.VMEM((1, B), jnp.int32),
            pltpu.VMEM((1, B), jnp.int32),
            pltpu.VMEM((1, B), jnp.int32),
            pltpu.VMEM((B, D), jnp.float32),
            pltpu.VMEM((B, D), jnp.float32),
            pltpu.VMEM((B, D), jnp.float32),
            pltpu.VMEM((B, D), jnp.float32),
            pltpu.VMEM_SHARED((N, D), jnp.float32),
            pltpu.SemaphoreType.DMA,
            pltpu.SemaphoreType.DMA,
            pltpu.SemaphoreType.DMA,
            pltpu.SemaphoreType.DMA,
            pltpu.SemaphoreType.DMA,
            pltpu.SemaphoreType.DMA,
        ],
    )
    def k(h_hbm, e_hbm, src_hbm, dst_hbm, out_hbm,
          si0, si1, di0, di1, eb0, eb1, gb0, gb1, acc,
          ssem0, ssem1, dsem0, dsem1, esem0, esem1):
        cid = lax.axis_index("c")
        sid = lax.axis_index("s")
        wid = sid * NC + cid
        ebase = wid * EPW
        r0 = sid * RPT

        sibufs = (si0, si1)
        dibufs = (di0, di1)
        ebufs = (eb0, eb1)
        gbufs = (gb0, gb1)
        sisems = (ssem0, ssem1)
        disems = (dsem0, dsem1)
        esems = (esem0, esem1)

        @pl.when(sid < NS - 1)
        def _():
            pltpu.sync_copy(h_hbm.at[pl.ds(r0, RPT)], acc.at[pl.ds(r0, RPT)])

        @pl.when(sid == NS - 1)
        def _():
            pltpu.sync_copy(h_hbm.at[pl.ds((NS - 1) * RPT, N - (NS - 1) * RPT)],
                            acc.at[pl.ds((NS - 1) * RPT, N - (NS - 1) * RPT)])

        plsc.subcore_barrier()

        def issue_si(blk, ph):
            pltpu.async_copy(
                src_hbm.at[wid, pl.ds(blk, 1)], sibufs[ph], sisems[ph])

        def wait_si(blk, ph):
            pltpu.make_async_copy(
                src_hbm.at[wid, pl.ds(blk, 1)], sibufs[ph], sisems[ph]).wait()

        def issue_de(blk, ph):
            pltpu.async_copy(
                dst_hbm.at[wid, pl.ds(blk, 1)], dibufs[ph], disems[ph])
            pltpu.async_copy(
                e_hbm.at[pl.ds(ebase + blk * B, B)], ebufs[ph], esems[ph])

        def issue_gather(blk, ph):
            pltpu.async_copy(h_hbm.at[sibufs[ph].at[0]], gbufs[ph], sisems[ph])

        def wait_data(blk, ph):
            pltpu.make_async_copy(
                e_hbm.at[pl.ds(ebase + blk * B, B)], ebufs[ph], esems[ph]).wait()
            pltpu.make_async_copy(
                h_hbm.at[sibufs[ph].at[0]], gbufs[ph], sisems[ph]).wait()

        def compute(ph):
            eb, gb = ebufs[ph], gbufs[ph]

            @pl.loop(0, B, step=2)
            def _(r):
                for rr in range(2):
                    for k8 in range(D // L):
                        sl = pl.ds(k8 * L, L)
                        eb[r + rr, sl] = jnp.maximum(
                            eb[r + rr, sl] + gb[r + rr, sl], 0.0)

        def scatter(blk, ph):
            pltpu.make_async_copy(
                dst_hbm.at[wid, pl.ds(blk, 1)], dibufs[ph], disems[ph]).wait()
            pltpu.sync_copy(ebufs[ph], acc.at[dibufs[ph].at[0]], add=True)

        issue_si(0, 0)
        issue_de(0, 0)
        issue_de(1, 1)
        wait_si(0, 0)
        issue_gather(0, 0)
        issue_si(1, 1)

        @pl.loop(0, NBLK - 1, step=2)
        def _(blk0):
            for ph in range(2):
                blk = blk0 + ph
                php = 1 - ph
                wait_data(blk, ph)

                @pl.when(blk + 2 < NBLK)
                def _():
                    issue_si(blk + 2, ph)

                wait_si(blk + 1, php)
                issue_gather(blk + 1, php)

                compute(ph)
                scatter(blk, ph)

                @pl.when(blk + 2 < NBLK)
                def _():
                    issue_de(blk + 2, ph)

        wait_data(NBLK - 1, (NBLK - 1) % 2)
        compute((NBLK - 1) % 2)
        scatter(NBLK - 1, (NBLK - 1) % 2)

        plsc.subcore_barrier()

        @pl.when(sid < NS - 1)
        def _():
            pltpu.sync_copy(acc.at[pl.ds(r0, RPT)],
                            out_hbm.at[cid, pl.ds(r0, RPT)])

        @pl.when(sid == NS - 1)
        def _():
            pltpu.sync_copy(acc.at[pl.ds((NS - 1) * RPT, N - (NS - 1) * RPT)],
                            out_hbm.at[cid, pl.ds((NS - 1) * RPT,
                                                  N - (NS - 1) * RPT)])

    return k(h, e, src2, dst2)


def kernel(x, edge_index, edge_weight, Wp, bp, We0, be0, W01, b01, W02, b02,
           We1, be1, W11, b11, W12, b12, Wpost, bpost):
    src2 = edge_index[0].astype(jnp.int32).reshape(NW, NBLK, B)
    dst2 = edge_index[1].astype(jnp.int32).reshape(NW, NBLK, B)

    h = _prep(x, Wp, bp)
    E0 = _edge_lin(edge_weight, We0, be0)
    E1 = _edge_lin(edge_weight, We1, be1)

    P0 = _sc_aggr(h, E0, src2, dst2)
    h1 = _mlp(h, P0, W01, b01, W02, b02)

    P1 = _sc_aggr(h1, E1, src2, dst2)
    out = _mlp_post(h1, P1, W11, b11, W12, b12, Wpost, bpost)
    return out


# speedup vs baseline: 4.9916x; 1.0463x over previous
import functools

import jax
import jax.numpy as jnp
from jax import lax
from jax.experimental import pallas as pl
from jax.experimental.pallas import tpu as pltpu
from jax.experimental.pallas import tpu_sc as plsc

N = 10000
E = 320000
D = 128
DE = 16
NEG = 0.2

NC, NS, L = 2, 16, 16
NW = NC * NS
EPW = E // NW
B = 80
NBLK = EPW // B
RPT = 624

_PREC = None


def _leaky(z):
    return jnp.maximum(z, 0.0) + NEG * jnp.minimum(z, 0.0)


def _dot(a, b):
    return jnp.dot(a, b, precision=_PREC, preferred_element_type=jnp.float32)


def _prep_body(x_ref, w_ref, b_ref, o_ref):
    o_ref[...] = _leaky(_dot(x_ref[...], w_ref[...]) + b_ref[...])


def _prep(x, Wp, bp):
    return pl.pallas_call(
        _prep_body,
        grid=(10,),
        in_specs=[
            pl.BlockSpec((N // 10, D), lambda i: (i, 0)),
            pl.BlockSpec((D, D), lambda i: (0, 0)),
            pl.BlockSpec((1, D), lambda i: (0, 0)),
        ],
        out_specs=pl.BlockSpec((N // 10, D), lambda i: (i, 0)),
        out_shape=jax.ShapeDtypeStruct((N, D), jnp.float32),
    )(x, Wp, bp.reshape(1, D))


def _edge_body(ew_ref, w_ref, b_ref, o_ref):
    o_ref[...] = _dot(ew_ref[...], w_ref[...]) + b_ref[...]


def _edge_lin(ew, We, be):
    EB = 8000
    return pl.pallas_call(
        _edge_body,
        grid=(E // EB,),
        in_specs=[
            pl.BlockSpec((EB, DE), lambda i: (i, 0)),
            pl.BlockSpec((DE, D), lambda i: (0, 0)),
            pl.BlockSpec((1, D), lambda i: (0, 0)),
        ],
        out_specs=pl.BlockSpec((EB, D), lambda i: (i, 0)),
        out_shape=jax.ShapeDtypeStruct((E, D), jnp.float32),
    )(ew, We, be.reshape(1, D))


def _mlp_body(h_ref, p0_ref, p1_ref, w1_ref, b1_ref, w2_ref, b2_ref, o_ref):
    z = p0_ref[0] + p1_ref[0] - h_ref[...]
    a = _leaky(_dot(z, w1_ref[...]) + b1_ref[...])
    o_ref[...] = jnp.tanh(_dot(a, w2_ref[...]) + b2_ref[...])


def _mlp(h, P, W1, b1, W2, b2):
    NB = N // 10
    return pl.pallas_call(
        _mlp_body,
        grid=(10,),
        in_specs=[
            pl.BlockSpec((NB, D), lambda i: (i, 0)),
            pl.BlockSpec((1, NB, D), lambda i: (0, i, 0)),
            pl.BlockSpec((1, NB, D), lambda i: (1, i, 0)),
            pl.BlockSpec((D, D), lambda i: (0, 0)),
            pl.BlockSpec((1, D), lambda i: (0, 0)),
            pl.BlockSpec((D, D), lambda i: (0, 0)),
            pl.BlockSpec((1, D), lambda i: (0, 0)),
        ],
        out_specs=pl.BlockSpec((NB, D), lambda i: (i, 0)),
        out_shape=jax.ShapeDtypeStruct((N, D), jnp.float32),
    )(h, P, P, W1, b1.reshape(1, D), W2, b2.reshape(1, D))


def _mlp_post_body(h_ref, p0_ref, p1_ref, w1_ref, b1_ref, w2_ref, b2_ref,
                   wp_ref, bp_ref, o_ref):
    z = p0_ref[0] + p1_ref[0] - h_ref[...]
    a = _leaky(_dot(z, w1_ref[...]) + b1_ref[...])
    t = jnp.tanh(_dot(a, w2_ref[...]) + b2_ref[...])
    o_ref[...] = jnp.tanh(_dot(t, wp_ref[...]) + bp_ref[...])


def _mlp_post(h, P, W1, b1, W2, b2, Wpost, bpost):
    NB = N // 10
    return pl.pallas_call(
        _mlp_post_body,
        grid=(10,),
        in_specs=[
            pl.BlockSpec((NB, D), lambda i: (i, 0)),
            pl.BlockSpec((1, NB, D), lambda i: (0, i, 0)),
            pl.BlockSpec((1, NB, D), lambda i: (1, i, 0)),
            pl.BlockSpec((D, D), lambda i: (0, 0)),
            pl.BlockSpec((1, D), lambda i: (0, 0)),
            pl.BlockSpec((D, D), lambda i: (0, 0)),
            pl.BlockSpec((1, D), lambda i: (0, 0)),
            pl.BlockSpec((D, D), lambda i: (0, 0)),
            pl.BlockSpec((1, D), lambda i: (0, 0)),
        ],
        out_specs=pl.BlockSpec((NB, D), lambda i: (i, 0)),
        out_shape=jax.ShapeDtypeStruct((N, D), jnp.float32),
    )(h, P, P, W1, b1.reshape(1, D), W2, b2.reshape(1, D), Wpost,
      bpost.reshape(1, D))


def _sc_aggr(h, e, src2, dst2):
    mesh = plsc.VectorSubcoreMesh(core_axis_name="c", subcore_axis_name="s")

    @functools.partial(
        pl.kernel,
        out_type=jax.ShapeDtypeStruct((NC, N, D), jnp.float32),
        mesh=mesh,
        scratch_types=[
            pltpu.VMEM((1, B), jnp.int32),
            pltpu.VMEM((1, B), jnp.int32),
            pltpu.VMEM((1, B), jnp.int32),
            pltpu.VMEM((1, B), jnp.int32),
            pltpu.VMEM((B, D), jnp.float32),
            pltpu.VMEM((B, D), jnp.float32),
            pltpu.VMEM((B, D), jnp.float32),
            pltpu.VMEM((B, D), jnp.float32),
            pltpu.VMEM_SHARED((N, D), jnp.float32),
            pltpu.SemaphoreType.DMA,
            pltpu.SemaphoreType.DMA,
            pltpu.SemaphoreType.DMA,
            pltpu.SemaphoreType.DMA,
            pltpu.SemaphoreType.DMA,
            pltpu.SemaphoreType.DMA,
            pltpu.SemaphoreType.DMA,
            pltpu.SemaphoreType.DMA,
        ],
    )
    def k(h_hbm, e_hbm, src_hbm, dst_hbm, out_hbm,
          si0, si1, di0, di1, eb0, eb1, gb0, gb1, acc,
          ssem0, ssem1, dsem0, dsem1, esem0, esem1, csem0, csem1):
        cid = lax.axis_index("c")
        sid = lax.axis_index("s")
        wid = sid * NC + cid
        ebase = wid * EPW
        r0 = sid * RPT

        sibufs = (si0, si1)
        dibufs = (di0, di1)
        ebufs = (eb0, eb1)
        gbufs = (gb0, gb1)
        sisems = (ssem0, ssem1)
        disems = (dsem0, dsem1)
        esems = (esem0, esem1)
        ssems = (csem0, csem1)

        @pl.when(sid < NS - 1)
        def _():
            pltpu.sync_copy(h_hbm.at[pl.ds(r0, RPT)], acc.at[pl.ds(r0, RPT)])

        @pl.when(sid == NS - 1)
        def _():
            pltpu.sync_copy(h_hbm.at[pl.ds((NS - 1) * RPT, N - (NS - 1) * RPT)],
                            acc.at[pl.ds((NS - 1) * RPT, N - (NS - 1) * RPT)])

        plsc.subcore_barrier()

        def issue_si(blk, ph):
            pltpu.async_copy(
                src_hbm.at[wid, pl.ds(blk, 1)], sibufs[ph], sisems[ph])

        def wait_si(blk, ph):
            pltpu.make_async_copy(
                src_hbm.at[wid, pl.ds(blk, 1)], sibufs[ph], sisems[ph]).wait()

        def issue_di(blk, ph):
            pltpu.async_copy(
                dst_hbm.at[wid, pl.ds(blk, 1)], dibufs[ph], disems[ph])

        def issue_e(blk, ph):
            pltpu.async_copy(
                e_hbm.at[pl.ds(ebase + blk * B, B)], ebufs[ph], esems[ph])

        def issue_gather(blk, ph):
            pltpu.async_copy(h_hbm.at[sibufs[ph].at[0]], gbufs[ph], sisems[ph])

        def wait_data(blk, ph):
            pltpu.make_async_copy(
                e_hbm.at[pl.ds(ebase + blk * B, B)], ebufs[ph], esems[ph]).wait()
            pltpu.make_async_copy(
                h_hbm.at[sibufs[ph].at[0]], gbufs[ph], sisems[ph]).wait()

        def compute(ph):
            eb, gb = ebufs[ph], gbufs[ph]

            @pl.loop(0, B, step=4)
            def _(r):
                for rr in range(4):
                    for k8 in range(D // L):
                        sl = pl.ds(k8 * L, L)
                        gb[r + rr, sl] = jnp.maximum(
                            eb[r + rr, sl] + gb[r + rr, sl], 0.0)

        def issue_scatter(blk, ph):
            pltpu.make_async_copy(
                dst_hbm.at[wid, pl.ds(blk, 1)], dibufs[ph], disems[ph]).wait()
            pltpu.async_copy(
                gbufs[ph], acc.at[dibufs[ph].at[0]], ssems[ph], add=True)

        def wait_scatter(ph):
            pltpu.make_async_copy(
                gbufs[ph], acc.at[dibufs[ph].at[0]], ssems[ph]).wait()

        issue_si(0, 0)
        issue_di(0, 0)
        issue_e(0, 0)
        issue_e(1, 1)
        wait_si(0, 0)
        issue_gather(0, 0)
        issue_si(1, 1)

        @pl.loop(0, NBLK - 1, step=2)
        def _(blk0):
            for ph in range(2):
                blk = blk0 + ph
                php = 1 - ph
                wait_data(blk, ph)

                @pl.when(blk + 2 < NBLK)
                def _():
                    issue_si(blk + 2, ph)

                @pl.when(blk >= 1)
                def _():
                    wait_scatter(php)

                wait_si(blk + 1, php)
                issue_gather(blk + 1, php)
                issue_di(blk + 1, php)

                compute(ph)

                @pl.when(blk + 2 < NBLK)
                def _():
                    issue_e(blk + 2, ph)

                issue_scatter(blk, ph)

        ph_t = (NBLK - 1) % 2
        wait_data(NBLK - 1, ph_t)
        wait_scatter(1 - ph_t)
        compute(ph_t)
        issue_scatter(NBLK - 1, ph_t)
        wait_scatter(ph_t)

        plsc.subcore_barrier()

        @pl.when(sid < NS - 1)
        def _():
            pltpu.sync_copy(acc.at[pl.ds(r0, RPT)],
                            out_hbm.at[cid, pl.ds(r0, RPT)])

        @pl.when(sid == NS - 1)
        def _():
            pltpu.sync_copy(acc.at[pl.ds((NS - 1) * RPT, N - (NS - 1) * RPT)],
                            out_hbm.at[cid, pl.ds((NS - 1) * RPT,
                                                  N - (NS - 1) * RPT)])

    return k(h, e, src2, dst2)


def kernel(x, edge_index, edge_weight, Wp, bp, We0, be0, W01, b01, W02, b02,
           We1, be1, W11, b11, W12, b12, Wpost, bpost):
    src2 = edge_index[0].astype(jnp.int32).reshape(NW, NBLK, B)
    dst2 = edge_index[1].astype(jnp.int32).reshape(NW, NBLK, B)

    h = _prep(x, Wp, bp)
    E0 = _edge_lin(edge_weight, We0, be0)
    E1 = _edge_lin(edge_weight, We1, be1)

    P0 = _sc_aggr(h, E0, src2, dst2)
    h1 = _mlp(h, P0, W01, b01, W02, b02)

    P1 = _sc_aggr(h1, E1, src2, dst2)
    out = _mlp_post(h1, P1, W11, b11, W12, b12, Wpost, bpost)
    return out


# speedup vs baseline: 5.1862x; 1.0390x over previous
import functools

import jax
import jax.numpy as jnp
from jax import lax
from jax.experimental import pallas as pl
from jax.experimental.pallas import tpu as pltpu
from jax.experimental.pallas import tpu_sc as plsc

N = 10000
E = 320000
D = 128
DE = 16
NEG = 0.2

NC, NS, L = 2, 16, 16
NW = NC * NS
EPW = E // NW
B = 80
NBLK = EPW // B
RPT = 624

_PREC = None


def _leaky(z):
    return jnp.maximum(z, 0.0) + NEG * jnp.minimum(z, 0.0)


def _dot(a, b):
    return jnp.dot(a, b, precision=_PREC, preferred_element_type=jnp.float32)


def _front_body(x_ref, wp_ref, bp_ref, ew_ref, w0_ref, b0_ref, w1_ref, b1_ref,
                h_ref, e0_ref, e1_ref):
    h_ref[...] = _leaky(_dot(x_ref[...], wp_ref[...]) + bp_ref[...])
    e0_ref[...] = _dot(ew_ref[...], w0_ref[...]) + b0_ref[...]
    e1_ref[...] = _dot(ew_ref[...], w1_ref[...]) + b1_ref[...]


def _front(x, Wp, bp, ew, We0, be0, We1, be1):
    EB = E // 40
    NB = N // 10
    full = lambda i: (0, 0)
    return pl.pallas_call(
        _front_body,
        grid=(40,),
        in_specs=[
            pl.BlockSpec((NB, D), lambda i: (i % 10, 0)),
            pl.BlockSpec((D, D), full),
            pl.BlockSpec((1, D), full),
            pl.BlockSpec((EB, DE), lambda i: (i, 0)),
            pl.BlockSpec((DE, D), full),
            pl.BlockSpec((1, D), full),
            pl.BlockSpec((DE, D), full),
            pl.BlockSpec((1, D), full),
        ],
        out_specs=[
            pl.BlockSpec((NB, D), lambda i: (i % 10, 0)),
            pl.BlockSpec((EB, D), lambda i: (i, 0)),
            pl.BlockSpec((EB, D), lambda i: (i, 0)),
        ],
        out_shape=[
            jax.ShapeDtypeStruct((N, D), jnp.float32),
            jax.ShapeDtypeStruct((E, D), jnp.float32),
            jax.ShapeDtypeStruct((E, D), jnp.float32),
        ],
    )(x, Wp, bp.reshape(1, D), ew, We0, be0.reshape(1, D), We1,
      be1.reshape(1, D))


def _mlp_body(h_ref, p0_ref, p1_ref, w1_ref, b1_ref, w2_ref, b2_ref, o_ref):
    z = p0_ref[0] + p1_ref[0] - h_ref[...]
    a = _leaky(_dot(z, w1_ref[...]) + b1_ref[...])
    o_ref[...] = jnp.tanh(_dot(a, w2_ref[...]) + b2_ref[...])


def _mlp(h, P, W1, b1, W2, b2):
    NB = N // 10
    return pl.pallas_call(
        _mlp_body,
        grid=(10,),
        in_specs=[
            pl.BlockSpec((NB, D), lambda i: (i, 0)),
            pl.BlockSpec((1, NB, D), lambda i: (0, i, 0)),
            pl.BlockSpec((1, NB, D), lambda i: (1, i, 0)),
            pl.BlockSpec((D, D), lambda i: (0, 0)),
            pl.BlockSpec((1, D), lambda i: (0, 0)),
            pl.BlockSpec((D, D), lambda i: (0, 0)),
            pl.BlockSpec((1, D), lambda i: (0, 0)),
        ],
        out_specs=pl.BlockSpec((NB, D), lambda i: (i, 0)),
        out_shape=jax.ShapeDtypeStruct((N, D), jnp.float32),
    )(h, P, P, W1, b1.reshape(1, D), W2, b2.reshape(1, D))


def _mlp_post_body(h_ref, p0_ref, p1_ref, w1_ref, b1_ref, w2_ref, b2_ref,
                   wp_ref, bp_ref, o_ref):
    z = p0_ref[0] + p1_ref[0] - h_ref[...]
    a = _leaky(_dot(z, w1_ref[...]) + b1_ref[...])
    t = jnp.tanh(_dot(a, w2_ref[...]) + b2_ref[...])
    o_ref[...] = jnp.tanh(_dot(t, wp_ref[...]) + bp_ref[...])


def _mlp_post(h, P, W1, b1, W2, b2, Wpost, bpost):
    NB = N // 10
    return pl.pallas_call(
        _mlp_post_body,
        grid=(10,),
        in_specs=[
            pl.BlockSpec((NB, D), lambda i: (i, 0)),
            pl.BlockSpec((1, NB, D), lambda i: (0, i, 0)),
            pl.BlockSpec((1, NB, D), lambda i: (1, i, 0)),
            pl.BlockSpec((D, D), lambda i: (0, 0)),
            pl.BlockSpec((1, D), lambda i: (0, 0)),
            pl.BlockSpec((D, D), lambda i: (0, 0)),
            pl.BlockSpec((1, D), lambda i: (0, 0)),
            pl.BlockSpec((D, D), lambda i: (0, 0)),
            pl.BlockSpec((1, D), lambda i: (0, 0)),
        ],
        out_specs=pl.BlockSpec((NB, D), lambda i: (i, 0)),
        out_shape=jax.ShapeDtypeStruct((N, D), jnp.float32),
    )(h, P, P, W1, b1.reshape(1, D), W2, b2.reshape(1, D), Wpost,
      bpost.reshape(1, D))


def _sc_aggr(h, e, src2, dst2):
    mesh = plsc.VectorSubcoreMesh(core_axis_name="c", subcore_axis_name="s")

    @functools.partial(
        pl.kernel,
        out_type=jax.ShapeDtypeStruct((NC, N, D), jnp.float32),
        mesh=mesh,
        scratch_types=[
            pltpu.VMEM((1, B), jnp.int32),
            pltpu.VMEM((1, B), jnp.int32),
            pltpu.VMEM((1, B), jnp.int32),
            pltpu.VMEM((1, B), jnp.int32),
            pltpu.VMEM((B, D), jnp.float32),
            pltpu.VMEM((B, D), jnp.float32),
            pltpu.VMEM((B, D), jnp.float32),
            pltpu.VMEM((B, D), jnp.float32),
            pltpu.VMEM_SHARED((N, D), jnp.float32),
            pltpu.SemaphoreType.DMA,
            pltpu.SemaphoreType.DMA,
            pltpu.SemaphoreType.DMA,
            pltpu.SemaphoreType.DMA,
            pltpu.SemaphoreType.DMA,
            pltpu.SemaphoreType.DMA,
            pltpu.SemaphoreType.DMA,
            pltpu.SemaphoreType.DMA,
        ],
    )
    def k(h_hbm, e_hbm, src_hbm, dst_hbm, out_hbm,
          si0, si1, di0, di1, eb0, eb1, gb0, gb1, acc,
          ssem0, ssem1, dsem0, dsem1, esem0, esem1, csem0, csem1):
        cid = lax.axis_index("c")
        sid = lax.axis_index("s")
        wid = sid * NC + cid
        ebase = wid * EPW
        r0 = sid * RPT

        sibufs = (si0, si1)
        dibufs = (di0, di1)
        ebufs = (eb0, eb1)
        gbufs = (gb0, gb1)
        sisems = (ssem0, ssem1)
        disems = (dsem0, dsem1)
        esems = (esem0, esem1)
        ssems = (csem0, csem1)

        @pl.when(sid < NS - 1)
        def _():
            pltpu.sync_copy(h_hbm.at[pl.ds(r0, RPT)], acc.at[pl.ds(r0, RPT)])

        @pl.when(sid == NS - 1)
        def _():
            pltpu.sync_copy(h_hbm.at[pl.ds((NS - 1) * RPT, N - (NS - 1) * RPT)],
                            acc.at[pl.ds((NS - 1) * RPT, N - (NS - 1) * RPT)])

        plsc.subcore_barrier()

        def issue_si(blk, ph):
            pltpu.async_copy(
                src_hbm.at[wid, pl.ds(blk, 1)], sibufs[ph], sisems[ph])

        def wait_si(blk, ph):
            pltpu.make_async_copy(
                src_hbm.at[wid, pl.ds(blk, 1)], sibufs[ph], sisems[ph]).wait()

        def issue_di(blk, ph):
            pltpu.async_copy(
                dst_hbm.at[wid, pl.ds(blk, 1)], dibufs[ph], disems[ph])

        def issue_e(blk, ph):
            pltpu.async_copy(
                e_hbm.at[pl.ds(ebase + blk * B, B)], ebufs[ph], esems[ph])

        def issue_gather(blk, ph):
            pltpu.async_copy(h_hbm.at[sibufs[ph].at[0]], gbufs[ph], sisems[ph])

        def wait_data(blk, ph):
            pltpu.make_async_copy(
                e_hbm.at[pl.ds(ebase + blk * B, B)], ebufs[ph], esems[ph]).wait()
            pltpu.make_async_copy(
                h_hbm.at[sibufs[ph].at[0]], gbufs[ph], sisems[ph]).wait()

        def compute(ph):
            eb, gb = ebufs[ph], gbufs[ph]

            @pl.loop(0, B, step=4)
            def _(r):
                for rr in range(4):
                    for k8 in range(D // L):
                        sl = pl.ds(k8 * L, L)
                        gb[r + rr, sl] = jnp.maximum(
                            eb[r + rr, sl] + gb[r + rr, sl], 0.0)

        def issue_scatter(blk, ph):
            pltpu.make_async_copy(
                dst_hbm.at[wid, pl.ds(blk, 1)], dibufs[ph], disems[ph]).wait()
            pltpu.async_copy(
                gbufs[ph], acc.at[dibufs[ph].at[0]], ssems[ph], add=True)

        def wait_scatter(ph):
            pltpu.make_async_copy(
                gbufs[ph], acc.at[dibufs[ph].at[0]], ssems[ph]).wait()

        issue_si(0, 0)
        issue_di(0, 0)
        issue_e(0, 0)
        issue_e(1, 1)
        wait_si(0, 0)
        issue_gather(0, 0)
        issue_si(1, 1)

        @pl.loop(0, NBLK - 1, step=2)
        def _(blk0):
            for ph in range(2):
                blk = blk0 + ph
                php = 1 - ph
                wait_data(blk, ph)

                @pl.when(blk + 2 < NBLK)
                def _():
                    issue_si(blk + 2, ph)

                @pl.when(blk >= 1)
                def _():
                    wait_scatter(php)

                wait_si(blk + 1, php)
                issue_gather(blk + 1, php)
                issue_di(blk + 1, php)

                compute(ph)

                @pl.when(blk + 2 < NBLK)
                def _():
                    issue_e(blk + 2, ph)

                issue_scatter(blk, ph)

        ph_t = (NBLK - 1) % 2
        wait_data(NBLK - 1, ph_t)
        wait_scatter(1 - ph_t)
        compute(ph_t)
        issue_scatter(NBLK - 1, ph_t)
        wait_scatter(ph_t)

        plsc.subcore_barrier()

        @pl.when(sid < NS - 1)
        def _():
            pltpu.sync_copy(acc.at[pl.ds(r0, RPT)],
                            out_hbm.at[cid, pl.ds(r0, RPT)])

        @pl.when(sid == NS - 1)
        def _():
            pltpu.sync_copy(acc.at[pl.ds((NS - 1) * RPT, N - (NS - 1) * RPT)],
                            out_hbm.at[cid, pl.ds((NS - 1) * RPT,
                                                  N - (NS - 1) * RPT)])

    return k(h, e, src2, dst2)


def kernel(x, edge_index, edge_weight, Wp, bp, We0, be0, W01, b01, W02, b02,
           We1, be1, W11, b11, W12, b12, Wpost, bpost):
    src2 = edge_index[0].astype(jnp.int32).reshape(NW, NBLK, B)
    dst2 = edge_index[1].astype(jnp.int32).reshape(NW, NBLK, B)

    h, E0, E1 = _front(x, Wp, bp, edge_weight, We0, be0, We1, be1)

    P0 = _sc_aggr(h, E0, src2, dst2)
    h1 = _mlp(h, P0, W01, b01, W02, b02)

    P1 = _sc_aggr(h1, E1, src2, dst2)
    out = _mlp_post(h1, P1, W11, b11, W12, b12, Wpost, bpost)
    return out
